# baseline (device time: 20310 ns/iter reference)
import jax
import jax.numpy as jnp
from jax import lax
from jax.experimental import pallas as pl
from jax.experimental.pallas import tpu as pltpu

N_DEV = 32
SLICE = 256 // N_DEV

ORDER_H1 = [14, 10, 13, 11, 12, 6, 5, 15, 2, 3, 9, 4, 7, 16, 8, 1]
ORDER_H2 = [18, 22, 19, 21, 20, 26, 17, 27, 30, 29, 23, 28, 25, 24, 31]


def kernel(x, W1, W2):
    m, _ = x.shape
    _, n = W2.shape
    half = m // 2

    def body(x_ref, w1_ref, w2_ref, out_ref, xx_buf, acc_buf, recv1,
             red_buf, send_sems1, recv_sems1, send_sems2, recv_sems2):
        my_pos = lax.axis_index("i")

        barrier_sem = pltpu.get_barrier_semaphore()
        pl.semaphore_signal(
            barrier_sem, inc=1,
            device_id=(my_pos,), device_id_type=pl.DeviceIdType.MESH,
        )
        pl.semaphore_wait(barrier_sem, 1)

        def send_r1(d):
            t = (my_pos + d) % N_DEV
            rdma = pltpu.make_async_remote_copy(
                src_ref=acc_buf.at[pl.ds((d - 1) * SLICE, SLICE)],
                dst_ref=recv1.at[d - 1],
                send_sem=send_sems1.at[d - 1],
                recv_sem=recv_sems1.at[d - 1],
                device_id=(t,),
                device_id_type=pl.DeviceIdType.MESH,
            )
            rdma.start()
            return rdma

        xx_buf[:m, :] = x_ref[:, :]
        xx_buf[m:, :] = x_ref[:, :]
        start = SLICE * (my_pos + 1)

        h1 = jnp.maximum(
            jnp.dot(xx_buf[pl.ds(start, half), :], w1_ref[:, :],
                    preferred_element_type=jnp.float32),
            0.0,
        )
        acc_buf[:half, :] = jnp.dot(h1, w2_ref[:, :],
                                    preferred_element_type=jnp.float32)
        r1 = [send_r1(d) for d in ORDER_H1]

        h2 = jnp.maximum(
            jnp.dot(xx_buf[pl.ds(start + half, half), :], w1_ref[:, :],
                    preferred_element_type=jnp.float32),
            0.0,
        )
        acc_buf[half:, :] = jnp.dot(h2, w2_ref[:, :],
                                    preferred_element_type=jnp.float32)
        r1 += [send_r1(d) for d in ORDER_H2]

        for rdma in r1:
            rdma.wait_recv()

        red_buf[:, :] = (
            acc_buf[(N_DEV - 1) * SLICE:, :]
            + jnp.sum(recv1[:, :, :], axis=0)
        )

        r2 = []
        for d in ORDER_H1 + ORDER_H2:
            t = (my_pos + d) % N_DEV
            rdma = pltpu.make_async_remote_copy(
                src_ref=red_buf,
                dst_ref=out_ref.at[pl.ds(my_pos * SLICE, SLICE)],
                send_sem=send_sems2.at[d - 1],
                recv_sem=recv_sems2.at[d - 1],
                device_id=(t,),
                device_id_type=pl.DeviceIdType.MESH,
            )
            rdma.start()
            r2.append(rdma)
        out_ref[pl.ds(my_pos * SLICE, SLICE), :] = red_buf[:, :]
        for rdma in r2:
            rdma.wait_recv()

        for rdma in r1:
            rdma.wait_send()
        for rdma in r2:
            rdma.wait_send()

    return pl.pallas_call(
        body,
        out_shape=jax.ShapeDtypeStruct((m, n), jnp.float32),
        in_specs=[pl.BlockSpec(memory_space=pltpu.VMEM)] * 3,
        out_specs=pl.BlockSpec(memory_space=pltpu.VMEM),
        scratch_shapes=[
            pltpu.VMEM((2 * m, n), jnp.float32),
            pltpu.VMEM((m, n), jnp.float32),
            pltpu.VMEM((N_DEV - 1, SLICE, n), jnp.float32),
            pltpu.VMEM((SLICE, n), jnp.float32),
            pltpu.SemaphoreType.DMA((N_DEV - 1,)),
            pltpu.SemaphoreType.DMA((N_DEV - 1,)),
            pltpu.SemaphoreType.DMA((N_DEV - 1,)),
            pltpu.SemaphoreType.DMA((N_DEV - 1,)),
        ],
        compiler_params=pltpu.CompilerParams(collective_id=0),
    )(x, W1, W2)


# device time: 19465 ns/iter; 1.0434x vs baseline; 1.0434x over previous
import jax
import jax.numpy as jnp
from jax import lax
from jax.experimental import pallas as pl
from jax.experimental.pallas import tpu as pltpu

N_DEV = 32
SLICE = 256 // N_DEV

ORDER_Q = [
    [6, 5, 2, 3, 4, 7, 8, 1],
    [14, 10, 13, 11, 12, 15, 9, 16],
    [18, 22, 19, 21, 20, 17, 23, 24],
    [26, 27, 30, 29, 28, 25, 31],
]
ORDER_ALL = [d for q in ORDER_Q for d in q]


def kernel(x, W1, W2):
    m, _ = x.shape
    _, n = W2.shape
    half = m // 2

    def body(x2_ref, w1_ref, w2_ref, out_ref, acc_buf, recv1, red_buf,
             send_sems1, recv_sems1, send_sems2, recv_sems2):
        my_pos = lax.axis_index("i")

        barrier_sem = pltpu.get_barrier_semaphore()
        pl.semaphore_signal(
            barrier_sem, inc=1,
            device_id=(my_pos,), device_id_type=pl.DeviceIdType.MESH,
        )
        pl.semaphore_wait(barrier_sem, 1)

        def send_r1(d):
            t = (my_pos + d) % N_DEV
            rdma = pltpu.make_async_remote_copy(
                src_ref=acc_buf.at[pl.ds((d - 1) * SLICE, SLICE)],
                dst_ref=recv1.at[d - 1],
                send_sem=send_sems1.at[d - 1],
                recv_sem=recv_sems1.at[d - 1],
                device_id=(t,),
                device_id_type=pl.DeviceIdType.MESH,
            )
            rdma.start()
            return rdma

        quarter = m // 4
        r1 = []
        for q in range(4):
            xq = x2_ref[pl.ds(SLICE * (my_pos + 1) + q * quarter, quarter), :]
            hq = jnp.maximum(
                jnp.dot(xq, w1_ref[:, :],
                        preferred_element_type=jnp.float32),
                0.0,
            )
            acc_buf[q * quarter:(q + 1) * quarter, :] = jnp.dot(
                hq, w2_ref[:, :], preferred_element_type=jnp.float32)
            r1 += [send_r1(d) for d in ORDER_Q[q]]

        for rdma in r1:
            rdma.wait_recv()

        red_buf[:, :] = (
            acc_buf[(N_DEV - 1) * SLICE:, :]
            + jnp.sum(recv1[:, :, :], axis=0)
        )

        r2 = []
        for d in ORDER_ALL:
            t = (my_pos + d) % N_DEV
            rdma = pltpu.make_async_remote_copy(
                src_ref=red_buf,
                dst_ref=out_ref.at[pl.ds(my_pos * SLICE, SLICE)],
                send_sem=send_sems2.at[d - 1],
                recv_sem=recv_sems2.at[d - 1],
                device_id=(t,),
                device_id_type=pl.DeviceIdType.MESH,
            )
            rdma.start()
            r2.append(rdma)
        out_ref[pl.ds(my_pos * SLICE, SLICE), :] = red_buf[:, :]
        for rdma in r2:
            rdma.wait_recv()

        for rdma in r1:
            rdma.wait_send()
        for rdma in r2:
            rdma.wait_send()

    x2 = jnp.concatenate([x, x], axis=0)
    return pl.pallas_call(
        body,
        out_shape=jax.ShapeDtypeStruct((m, n), jnp.float32),
        in_specs=[pl.BlockSpec(memory_space=pltpu.VMEM)] * 3,
        out_specs=pl.BlockSpec(memory_space=pltpu.VMEM),
        scratch_shapes=[
            pltpu.VMEM((m, n), jnp.float32),
            pltpu.VMEM((N_DEV - 1, SLICE, n), jnp.float32),
            pltpu.VMEM((SLICE, n), jnp.float32),
            pltpu.SemaphoreType.DMA((N_DEV - 1,)),
            pltpu.SemaphoreType.DMA((N_DEV - 1,)),
            pltpu.SemaphoreType.DMA((N_DEV - 1,)),
            pltpu.SemaphoreType.DMA((N_DEV - 1,)),
        ],
        compiler_params=pltpu.CompilerParams(collective_id=0),
    )(x2, W1, W2)
